# baseline (speedup 1.0000x reference)
import functools
import jax
import jax.numpy as jnp
import numpy as np
from jax import lax
from jax.experimental import pallas as pl
from jax.experimental.pallas import tpu as pltpu
from jax.experimental.pallas import tpu_sc as plsc

_DELTA_VAR = 0.5
_DELTA_DIST = 1.5
_ALPHA = 1.0
_BETA = 1.0
_GAMMA = 0.001
_NC = 16
_F = 32
_P = 16384
_B = 8
_CHUNK = 1024
_QUARTER = _P // 4
_NCHUNKS = _QUARTER // _CHUNK
_SUMS = _NC * _F
_ACC = _SUMS + _NC + _NC


def _sc_body(x_hbm, t_hbm, means_out, cnt_out, var_out,
             xbuf, tbuf, acc, means_v, tmp, shared):
    c = lax.axis_index("c")
    s = lax.axis_index("s")
    b = c * 4 + s // 4
    q = s % 4
    ones = jnp.ones((16,), jnp.float32)

    for i in range(_ACC // 16):
        acc[pl.ds(i * 16, 16)] = jnp.zeros((16,), jnp.float32)

    for k in range(_NCHUNKS):
        off = q * _QUARTER + k * _CHUNK
        pltpu.sync_copy(x_hbm.at[b, :, pl.ds(off, _CHUNK)], xbuf)
        pltpu.sync_copy(t_hbm.at[b, pl.ds(off, _CHUNK)], tbuf)

        def body_a(j, carry):
            tv = tbuf[pl.ds(j * 16, 16)]
            plsc.addupdate_scatter(acc, [tv + _SUMS], ones)
            base = tv * _F
            for f in range(_F):
                xv = xbuf[f, pl.ds(j * 16, 16)]
                plsc.addupdate_scatter(acc, [base + f], xv)
            return carry

        lax.fori_loop(0, _CHUNK // 16, body_a, 0)

    pltpu.sync_copy(acc.at[pl.ds(0, _SUMS + _NC)],
                    shared.at[pl.ds(s * _ACC, _SUMS + _NC)])
    plsc.subcore_barrier()

    @pl.when(q == 0)
    def _reduce_a():
        for r in range(1, 4):
            pltpu.sync_copy(shared.at[pl.ds((s + r) * _ACC, _SUMS + _NC)],
                            tmp.at[pl.ds(0, _SUMS + _NC)])
            for i in range((_SUMS + _NC) // 16):
                sl = pl.ds(i * 16, 16)
                acc[sl] = acc[sl] + tmp[sl]
        safe_vec = jnp.maximum(acc[pl.ds(_SUMS, 16)], 1.0)
        for i in range(_SUMS // 16):
            cl = i // 2
            means_v[pl.ds(i * 16, 16)] = acc[pl.ds(i * 16, 16)] / safe_vec[cl]
        pltpu.sync_copy(means_v, shared.at[pl.ds(s * _ACC, _SUMS)])

    plsc.subcore_barrier()

    @pl.when(q != 0)
    def _fetch_means():
        pltpu.sync_copy(shared.at[pl.ds((s - q) * _ACC, _SUMS)], means_v)

    for k in range(_NCHUNKS):
        off = q * _QUARTER + k * _CHUNK
        pltpu.sync_copy(x_hbm.at[b, :, pl.ds(off, _CHUNK)], xbuf)
        pltpu.sync_copy(t_hbm.at[b, pl.ds(off, _CHUNK)], tbuf)

        def body_b(j, carry):
            tv = tbuf[pl.ds(j * 16, 16)]
            base = tv * _F
            dev = jnp.zeros((16,), jnp.float32)
            for f in range(_F):
                xv = xbuf[f, pl.ds(j * 16, 16)]
                mv = plsc.load_gather(means_v, [base + f])
                dev = dev + jnp.abs(xv - mv)
            hv = jnp.maximum(dev - _DELTA_VAR, 0.0)
            var = hv * hv
            plsc.addupdate_scatter(acc, [tv + _SUMS + _NC], var)
            return carry

        lax.fori_loop(0, _CHUNK // 16, body_b, 0)

    pltpu.sync_copy(acc.at[pl.ds(_SUMS + _NC, _NC)],
                    shared.at[pl.ds(s * _ACC + _SUMS + _NC, _NC)])
    plsc.subcore_barrier()

    @pl.when(q == 0)
    def _reduce_b():
        for r in range(1, 4):
            pltpu.sync_copy(shared.at[pl.ds((s + r) * _ACC + _SUMS + _NC, _NC)],
                            tmp.at[pl.ds(0, _NC)])
            sl = pl.ds(_SUMS + _NC, _NC)
            acc[sl] = acc[sl] + tmp[pl.ds(0, _NC)]
        pltpu.sync_copy(means_v, means_out.at[pl.ds(b * _SUMS, _SUMS)])
        pltpu.sync_copy(acc.at[pl.ds(_SUMS, _NC)], cnt_out.at[pl.ds(b * _NC, _NC)])
        pltpu.sync_copy(acc.at[pl.ds(_SUMS + _NC, _NC)], var_out.at[pl.ds(b * _NC, _NC)])


def _sc_stats(x, t):
    mesh = plsc.VectorSubcoreMesh(core_axis_name="c", subcore_axis_name="s")
    kfn = functools.partial(
        pl.kernel, mesh=mesh,
        out_type=(
            jax.ShapeDtypeStruct((_B * _SUMS,), jnp.float32),
            jax.ShapeDtypeStruct((_B * _NC,), jnp.float32),
            jax.ShapeDtypeStruct((_B * _NC,), jnp.float32),
        ),
        scratch_types=[
            pltpu.VMEM((_F, _CHUNK), jnp.float32),
            pltpu.VMEM((_CHUNK,), jnp.int32),
            pltpu.VMEM((_ACC,), jnp.float32),
            pltpu.VMEM((_SUMS,), jnp.float32),
            pltpu.VMEM((_SUMS + _NC,), jnp.float32),
            pltpu.VMEM_SHARED((16 * _ACC,), jnp.float32),
        ],
        compiler_params=pltpu.CompilerParams(needs_layout_passes=False),
    )(_sc_body)
    return kfn(x, t)


def _tail_kernel(means_ref, cnt_ref, vs_ref, t16_ref, rj_ref, tk_ref,
                 mg_ref, out_ref):
    means = means_ref[...]
    cnt = cnt_ref[...]
    vs = vs_ref[...]
    t16 = t16_ref[...]
    rj = rj_ref[...]
    tk = tk_ref[...]
    mg = mg_ref[...]

    safe = jnp.maximum(cnt, 1.0)
    present = cnt > 0.0
    presf = present.astype(jnp.float32)
    c_var = jnp.where(present, vs / safe, 0.0)
    Ks = jnp.sum(presf, axis=1, keepdims=True)

    norms = jnp.sum(jnp.abs(means), axis=1, keepdims=True)
    norms = norms.reshape(_B, _NC)
    regs = jnp.sum(jnp.where(present, norms, 0.0), axis=1, keepdims=True) / Ks

    total = _ALPHA * jnp.sum(c_var) + _GAMMA * jnp.sum(regs)
    for bb in range(_B):
        means_b = lax.slice(means, (bb * _NC, 0), ((bb + 1) * _NC, _F))
        t_row = lax.slice(t16, (bb, 0), (bb + 1, _NC))
        lbl_col = jax.lax.broadcasted_iota(jnp.int32, (_NC, 1), 0)
        onehot = (lbl_col == t_row).astype(jnp.float32)
        mc = jax.lax.dot_general(
            means_b, onehot, dimension_numbers=(((0,), (0,)), ((), ())),
            preferred_element_type=jnp.float32)
        mc_j = jax.lax.dot_general(
            mc, rj, dimension_numbers=(((1,), (0,)), ((), ())),
            preferred_element_type=jnp.float32)
        mc_k = jax.lax.dot_general(
            mc, tk, dimension_numbers=(((1,), (0,)), ((), ())),
            preferred_element_type=jnp.float32)
        d = jnp.sum(jnp.abs(mc_j - mc_k), axis=0, keepdims=True)
        pres_row = lax.slice(presf, (bb, 0), (bb + 1, _NC))
        pres_j = jax.lax.dot_general(
            pres_row, rj, dimension_numbers=(((1,), (0,)), ((), ())),
            preferred_element_type=jnp.float32)
        pres_k = jax.lax.dot_general(
            pres_row, tk, dimension_numbers=(((1,), (0,)), ((), ())),
            preferred_element_type=jnp.float32)
        hinge = jnp.maximum(mg - d, 0.0) ** 2
        c_dist = jnp.sum(pres_j * pres_k * hinge)
        K = Ks[bb, 0]
        denom = jnp.maximum(K * (K - 1.0), 1.0)
        total = total + _BETA * jnp.where(K > 1.0, c_dist / denom, 0.0)

    out_ref[...] = jnp.full((1, 1), total / _B, jnp.float32)


def _pair_constants():
    nc = _NC
    rj = np.zeros((nc, nc * nc), np.float32)
    tk = np.zeros((nc, nc * nc), np.float32)
    for j in range(nc):
        for k in range(nc):
            rj[j, j * nc + k] = 1.0
            tk[k, j * nc + k] = 1.0
    mg = np.full((1, nc * nc), 2.0 * _DELTA_DIST, np.float32)
    for j in range(nc):
        mg[0, j * nc + j] = 0.0
    return jnp.asarray(rj), jnp.asarray(tk), jnp.asarray(mg)


def kernel(input, target):
    means_r, cnt, vs = _sc_stats(input, target)
    means128 = means_r.reshape(_B * _NC, _F)
    cnt = cnt.reshape(_B, _NC)
    vs = vs.reshape(_B, _NC)
    t16 = target[:, :_NC]
    rj, tk, mg = _pair_constants()
    nn = _NC * _NC
    out = pl.pallas_call(
        _tail_kernel,
        in_specs=[
            pl.BlockSpec((_B * _NC, _F), lambda: (0, 0)),
            pl.BlockSpec((_B, _NC), lambda: (0, 0)),
            pl.BlockSpec((_B, _NC), lambda: (0, 0)),
            pl.BlockSpec((_B, _NC), lambda: (0, 0)),
            pl.BlockSpec((_NC, nn), lambda: (0, 0)),
            pl.BlockSpec((_NC, nn), lambda: (0, 0)),
            pl.BlockSpec((1, nn), lambda: (0, 0)),
        ],
        out_specs=pl.BlockSpec((1, 1), lambda: (0, 0)),
        out_shape=jax.ShapeDtypeStruct((1, 1), jnp.float32),
    )(means128, cnt, vs, t16, rj, tk, mg)
    return out[0, 0]


# speedup vs baseline: 12.2228x; 12.2228x over previous
import jax
import jax.numpy as jnp
import numpy as np
from jax.experimental import pallas as pl

_DELTA_VAR = 0.5
_DELTA_DIST = 1.5
_ALPHA = 1.0
_BETA = 1.0
_GAMMA = 0.001
_NC = 16


def _batch_contrib(x, t, rj, tk, mg, ones_p):
    lbl = jax.lax.broadcasted_iota(jnp.int32, (_NC, 1), 0)
    mask = (t == lbl).astype(jnp.float32)

    counts = jax.lax.dot_general(
        mask, ones_p, dimension_numbers=(((1,), (1,)), ((), ())),
        preferred_element_type=jnp.float32)
    safe_counts = jnp.maximum(counts, 1.0)
    present = counts > 0.0

    sums = jax.lax.dot_general(
        mask, x, dimension_numbers=(((1,), (1,)), ((), ())),
        preferred_element_type=jnp.float32)
    means = sums / safe_counts

    c_means = jax.lax.dot_general(
        means, mask, dimension_numbers=(((0,), (0,)), ((), ())),
        preferred_element_type=jnp.float32)

    dev = jnp.sum(jnp.abs(x - c_means), axis=0, keepdims=True)
    var = jnp.maximum(dev - _DELTA_VAR, 0.0) ** 2
    var_sums = jax.lax.dot_general(
        mask, var, dimension_numbers=(((1,), (1,)), ((), ())),
        preferred_element_type=jnp.float32)
    c_var = jnp.where(present, var_sums / safe_counts, 0.0)
    var_term = jnp.sum(c_var)

    mc = c_means[:, :_NC]
    mc_j = jax.lax.dot_general(
        mc, rj, dimension_numbers=(((1,), (0,)), ((), ())),
        preferred_element_type=jnp.float32)
    mc_k = jax.lax.dot_general(
        mc, tk, dimension_numbers=(((1,), (0,)), ((), ())),
        preferred_element_type=jnp.float32)
    d = jnp.sum(jnp.abs(mc_j - mc_k), axis=0, keepdims=True)
    presf = present.astype(jnp.float32)
    pres_j = jax.lax.dot_general(
        presf, rj, dimension_numbers=(((0,), (0,)), ((), ())),
        preferred_element_type=jnp.float32)
    pres_k = jax.lax.dot_general(
        presf, tk, dimension_numbers=(((0,), (0,)), ((), ())),
        preferred_element_type=jnp.float32)
    hinge = jnp.maximum(mg - d, 0.0) ** 2
    c_dist = jnp.sum(pres_j * pres_k * hinge)
    K = jnp.sum(presf)
    denom = jnp.maximum(K * (K - 1.0), 1.0)
    dist_term = jnp.where(K > 1.0, c_dist / denom, 0.0)

    col_norms = jnp.where(present, jnp.sum(jnp.abs(means), axis=1,
                                           keepdims=True), 0.0)
    reg_term = jnp.sum(col_norms) / K

    return (_ALPHA * var_term + _BETA * dist_term + _GAMMA * reg_term)


def _loss_kernel(x_ref, t_ref, rj_ref, tk_ref, mg_ref, out_ref):
    b = pl.program_id(0)
    nb = pl.num_programs(0)
    bpb = x_ref.shape[0]
    P = x_ref.shape[2]

    rj = rj_ref[...]
    tk = tk_ref[...]
    mg = mg_ref[...]
    ones_p = jnp.ones((1, P), jnp.float32)

    contrib = 0.0
    for bb in range(bpb):
        contrib = contrib + _batch_contrib(
            x_ref[bb], t_ref[bb], rj, tk, mg, ones_p)
    contrib = contrib / (nb * bpb)

    @pl.when(b == 0)
    def _():
        out_ref[...] = jnp.zeros((1, 1), jnp.float32)

    out_ref[...] += jnp.full((1, 1), contrib, jnp.float32)


def _pair_constants():
    nc = _NC
    rj = np.zeros((nc, nc * nc), np.float32)
    tk = np.zeros((nc, nc * nc), np.float32)
    for j in range(nc):
        for k in range(nc):
            rj[j, j * nc + k] = 1.0
            tk[k, j * nc + k] = 1.0
    mg = np.full((1, nc * nc), 2.0 * _DELTA_DIST, np.float32)
    for j in range(nc):
        mg[0, j * nc + j] = 0.0
    return jnp.asarray(rj), jnp.asarray(tk), jnp.asarray(mg)


def kernel(input, target):
    B, F, P = input.shape
    t3 = target.reshape(B, 1, P)
    rj, tk, mg = _pair_constants()
    nn = _NC * _NC
    bpb = 4 if B % 4 == 0 else 1
    out = pl.pallas_call(
        _loss_kernel,
        grid=(B // bpb,),
        in_specs=[
            pl.BlockSpec((bpb, F, P), lambda i: (i, 0, 0)),
            pl.BlockSpec((bpb, 1, P), lambda i: (i, 0, 0)),
            pl.BlockSpec((_NC, nn), lambda i: (0, 0)),
            pl.BlockSpec((_NC, nn), lambda i: (0, 0)),
            pl.BlockSpec((1, nn), lambda i: (0, 0)),
        ],
        out_specs=pl.BlockSpec((1, 1), lambda i: (0, 0)),
        out_shape=jax.ShapeDtypeStruct((1, 1), jnp.float32),
    )(input, t3, rj, tk, mg)
    return out[0, 0]
